# Initial kernel scaffold; baseline (speedup 1.0000x reference)
import jax
import jax.numpy as jnp
from jax.experimental import pallas as pl


def kernel(x, wcat, bcat, b0_w0, b0_b0, b0_w1, b0_b1, b1_w0, b1_b0, b1_w1, b1_b1, b2_w0, b2_b0, b2_w1, b2_b1, b2_w2, b2_b2, wl0, wl1, wl2, blin):
    raise NotImplementedError("write your pallas kernel here")


# speedup vs baseline: 1.0046x; 1.0046x over previous
import functools

import jax
import jax.numpy as jnp
from jax import lax
from jax.experimental import pallas as pl
from jax.experimental.pallas import tpu as pltpu


def _make_rfb_kernel(H, W, inter, scale, vision):
    L = H * W
    bf16 = jnp.bfloat16

    def body(x_ref, wcat, bcat,
             w01, b01, w02, b02,
             w11, b11, w12, b12,
             w21, b21, w22, b22, w23, b23,
             wl, blin,
             o_ref, colbuf, ybuf):
        idx = lax.broadcasted_iota(jnp.int32, (1, L), 1)
        row = idx // W
        col = lax.rem(idx, W)

        masks = {}

        def tap_mask(dy, dx):
            if (dy, dx) not in masks:
                conds = []
                if dy > 0:
                    conds.append(row < H - dy)
                elif dy < 0:
                    conds.append(row >= -dy)
                if dx > 0:
                    conds.append(col < W - dx)
                elif dx < 0:
                    conds.append(col >= -dx)
                m = conds[0]
                for c in conds[1:]:
                    m = m & c
                masks[(dy, dx)] = m
            return masks[(dy, dx)]

        def shift2d(x, dy, dx):
            if dy == 0 and dx == 0:
                return x
            s = (-(dy * W + dx)) % L
            return jnp.where(tap_mask(dy, dx), pltpu.roll(x, s, axis=1),
                             jnp.zeros((), bf16))

        def conv3x3(xb, w_ref, b_ref, dil, relu):
            cin = xb.shape[0]
            r = 0
            for ky in (-1, 0, 1):
                for kx in (-1, 0, 1):
                    colbuf[r:r + cin, :] = shift2d(xb, ky * dil, kx * dil)
                    r += cin
            y = jnp.dot(w_ref[...], colbuf[0:r, :],
                        preferred_element_type=jnp.float32) + b_ref[...]
            return jnp.maximum(y, 0.0) if relu else y

        xb = x_ref[0].astype(bf16)
        t = jnp.dot(wcat[...], xb,
                    preferred_element_type=jnp.float32) + bcat[...]
        i = inter
        z0 = t[0:i].astype(bf16)
        z1 = t[i:2 * i].astype(bf16)
        z2 = t[2 * i:3 * i].astype(bf16)
        short = t[3 * i:]

        co = wl.shape[1] // 3

        y0 = conv3x3(z0, w01, b01, 1, True).astype(bf16)
        ybuf[0:co, :] = conv3x3(y0, w02, b02, vision + 1, False).astype(bf16)
        y1 = conv3x3(z1, w11, b11, 1, True).astype(bf16)
        ybuf[co:2 * co, :] = conv3x3(y1, w12, b12, vision + 2, False).astype(bf16)
        y2 = conv3x3(z2, w21, b21, 1, True).astype(bf16)
        y2 = conv3x3(y2, w22, b22, 1, True).astype(bf16)
        ybuf[2 * co:3 * co, :] = conv3x3(y2, w23, b23, vision + 4, False).astype(bf16)

        out = jnp.dot(wl[...], ybuf[...],
                      preferred_element_type=jnp.float32) + blin[...]

        out = out * scale + short
        o_ref[0] = jnp.maximum(out, 0.0).astype(o_ref.dtype)

    return body


def _full_spec(a):
    zeros = (0,) * a.ndim
    return pl.BlockSpec(a.shape, lambda n, _z=zeros: _z)


@functools.partial(jax.jit, static_argnames=("scale", "vision"))
def _rfb_forward(x_nchw, ops, *, scale=0.1, vision=1):
    N, C1, H, W = x_nchw.shape
    L = H * W
    bf16 = jnp.bfloat16

    wcat, bcat = ops["wcat"], ops["bcat"]
    (w01, b01), (w02, b02) = ops["b0"]
    (w11, b11), (w12, b12) = ops["b1"]
    (w21, b21), (w22, b22), (w23, b23) = ops["b2"]
    wl, blin = ops["lin"]
    c2 = blin.shape[0]
    inter = (wcat.shape[0] - c2) // 3

    xf = x_nchw.reshape(N, C1, L)

    weight_list = [wcat.astype(bf16), bcat,
                   w01.astype(bf16), b01, w02.astype(bf16), b02,
                   w11.astype(bf16), b11, w12.astype(bf16), b12,
                   w21.astype(bf16), b21, w22.astype(bf16), b22,
                   w23.astype(bf16), b23,
                   wl.astype(bf16), blin]

    kernel_fn = _make_rfb_kernel(H, W, inter, scale, vision)

    rows = 9 * max(2 * inter, inter // 2 * 3, inter)
    rows = -(-rows // 16) * 16
    ycat = wl.shape[1]

    in_specs = [pl.BlockSpec((1, C1, L), lambda n: (n, 0, 0))]
    in_specs += [_full_spec(w) for w in weight_list]

    out = pl.pallas_call(
        kernel_fn,
        out_shape=jax.ShapeDtypeStruct((N, c2, L), x_nchw.dtype),
        grid=(N,),
        in_specs=in_specs,
        out_specs=pl.BlockSpec((1, c2, L), lambda n: (n, 0, 0)),
        scratch_shapes=[pltpu.VMEM((rows, L), bf16),
                        pltpu.VMEM((ycat, L), bf16)],
        compiler_params=pltpu.CompilerParams(dimension_semantics=("parallel",)),
    )(xf, *weight_list)
    return out.reshape(N, c2, H, W)


def kernel(x, wcat, bcat, b0_w0, b0_b0, b0_w1, b0_b1,
           b1_w0, b1_b0, b1_w1, b1_b1,
           b2_w0, b2_b0, b2_w1, b2_b1, b2_w2, b2_b2,
           wl0, wl1, wl2, blin):
    ops = {
        "wcat": wcat, "bcat": bcat,
        "b0": [(b0_w0, b0_b0), (b0_w1, b0_b1)],
        "b1": [(b1_w0, b1_b0), (b1_w1, b1_b1)],
        "b2": [(b2_w0, b2_b0), (b2_w1, b2_b1), (b2_w2, b2_b2)],
        "lin": (jnp.concatenate([wl0, wl1, wl2], axis=1), blin),
    }
    return _rfb_forward(x, ops, scale=0.1, vision=1)


# speedup vs baseline: 1.0505x; 1.0458x over previous
import functools

import jax
import jax.numpy as jnp
from jax import lax
from jax.experimental import pallas as pl
from jax.experimental.pallas import tpu as pltpu


def _make_rfb_kernel(H, W, inter, scale, vision, layout):
    L = H * W
    bf16 = jnp.bfloat16

    def body(x_ref, wall_ref, ball_ref, o_ref, colbuf, ybuf):
        idx = lax.broadcasted_iota(jnp.int32, (1, L), 1)
        row = idx // W
        col = lax.rem(idx, W)

        masks = {}

        def tap_mask(dy, dx):
            if (dy, dx) not in masks:
                conds = []
                if dy > 0:
                    conds.append(row < H - dy)
                elif dy < 0:
                    conds.append(row >= -dy)
                if dx > 0:
                    conds.append(col < W - dx)
                elif dx < 0:
                    conds.append(col >= -dx)
                m = conds[0]
                for c in conds[1:]:
                    m = m & c
                masks[(dy, dx)] = m
            return masks[(dy, dx)]

        def shift2d(x, dy, dx):
            if dy == 0 and dx == 0:
                return x
            s = (-(dy * W + dx)) % L
            return jnp.where(tap_mask(dy, dx), pltpu.roll(x, s, axis=1),
                             jnp.zeros((), bf16))

        def wb(name):
            r0, r1, k = layout[name]
            return wall_ref[r0:r1, 0:k], ball_ref[r0:r1, :]

        def conv3x3(xb, name, dil, relu):
            w, b = wb(name)
            cin = xb.shape[0]
            r = 0
            for ky in (-1, 0, 1):
                for kx in (-1, 0, 1):
                    colbuf[r:r + cin, :] = shift2d(xb, ky * dil, kx * dil)
                    r += cin
            y = jnp.dot(w, colbuf[0:r, :],
                        preferred_element_type=jnp.float32) + b
            return jnp.maximum(y, 0.0) if relu else y

        xb = x_ref[0].astype(bf16)
        wcat, bcat = wb("wcat")
        t = jnp.dot(wcat, xb, preferred_element_type=jnp.float32) + bcat
        i = inter
        z0 = t[0:i].astype(bf16)
        z1 = t[i:2 * i].astype(bf16)
        z2 = t[2 * i:3 * i].astype(bf16)
        short = t[3 * i:]

        co = 2 * inter

        y0 = conv3x3(z0, "w01", 1, True).astype(bf16)
        ybuf[0:co, :] = conv3x3(y0, "w02", vision + 1, False).astype(bf16)
        y1 = conv3x3(z1, "w11", 1, True).astype(bf16)
        ybuf[co:2 * co, :] = conv3x3(y1, "w12", vision + 2, False).astype(bf16)
        y2 = conv3x3(z2, "w21", 1, True).astype(bf16)
        y2 = conv3x3(y2, "w22", 1, True).astype(bf16)
        ybuf[2 * co:3 * co, :] = conv3x3(y2, "w23", vision + 4, False).astype(bf16)

        wl, blin = wb("wl")
        out = jnp.dot(wl, ybuf[...], preferred_element_type=jnp.float32) + blin

        out = out * scale + short
        o_ref[0] = jnp.maximum(out, 0.0).astype(o_ref.dtype)

    return body


@functools.partial(jax.jit, static_argnames=("scale", "vision"))
def _rfb_forward(x_nchw, weights, *, scale=0.1, vision=1):
    N, C1, H, W = x_nchw.shape
    L = H * W
    bf16 = jnp.bfloat16

    wcat = weights["wcat"]
    c2 = weights["blin"].shape[0]
    inter = (wcat.shape[0] - c2) // 3

    order = [("wcat", "bcat"), ("w01", "b01"), ("w02", "b02"),
             ("w11", "b11"), ("w12", "b12"), ("w21", "b21"),
             ("w22", "b22"), ("w23", "b23"), ("wl", "blin")]
    kmax = max(weights[wn].shape[1] for wn, _ in order)
    layout = {}
    wrows, brows = [], []
    r = 0
    for wn, bn in order:
        w, b = weights[wn], weights[bn]
        m, k = w.shape
        layout[wn] = (r, r + m, k)
        wrows.append(jnp.pad(w.astype(bf16), ((0, 0), (0, kmax - k))))
        brows.append(b)
        r += m
    wall = jnp.concatenate(wrows, axis=0)
    ball = jnp.concatenate(brows, axis=0)

    xf = x_nchw.reshape(N, C1, L)
    kernel_fn = _make_rfb_kernel(H, W, inter, scale, vision, layout)

    rows = 9 * max(2 * inter, inter // 2 * 3, inter)
    rows = -(-rows // 16) * 16

    out = pl.pallas_call(
        kernel_fn,
        out_shape=jax.ShapeDtypeStruct((N, c2, L), x_nchw.dtype),
        grid=(N,),
        in_specs=[pl.BlockSpec((1, C1, L), lambda n: (n, 0, 0)),
                  pl.BlockSpec(wall.shape, lambda n: (0, 0)),
                  pl.BlockSpec(ball.shape, lambda n: (0, 0))],
        out_specs=pl.BlockSpec((1, c2, L), lambda n: (n, 0, 0)),
        scratch_shapes=[pltpu.VMEM((rows, L), bf16),
                        pltpu.VMEM((3 * 2 * inter, L), bf16)],
        compiler_params=pltpu.CompilerParams(dimension_semantics=("parallel",)),
    )(xf, wall, ball)
    return out.reshape(N, c2, H, W)


def kernel(x, wcat, bcat, b0_w0, b0_b0, b0_w1, b0_b1,
           b1_w0, b1_b0, b1_w1, b1_b1,
           b2_w0, b2_b0, b2_w1, b2_b1, b2_w2, b2_b2,
           wl0, wl1, wl2, blin):
    weights = {
        "wcat": wcat, "bcat": bcat,
        "w01": b0_w0, "b01": b0_b0, "w02": b0_w1, "b02": b0_b1,
        "w11": b1_w0, "b11": b1_b0, "w12": b1_w1, "b12": b1_b1,
        "w21": b2_w0, "b21": b2_b0, "w22": b2_w1, "b22": b2_b1,
        "w23": b2_w2, "b23": b2_b2,
        "wl": jnp.concatenate([wl0, wl1, wl2], axis=1), "blin": blin,
    }
    return _rfb_forward(x, weights, scale=0.1, vision=1)


# speedup vs baseline: 1.1287x; 1.0744x over previous
import functools

import jax
import jax.numpy as jnp
from jax import lax
from jax.experimental import pallas as pl
from jax.experimental.pallas import tpu as pltpu


def _make_rfb_kernel(H, W, inter, scale, vision, wslots, bslots):
    L = H * W
    bf16 = jnp.bfloat16

    def body(x_ref, *args):
        nw = len(set(g for g, _, _ in wslots.values()))
        wrefs = args[:nw]
        ball_ref = args[nw]
        o_ref, colbuf, ybuf = args[nw + 1], args[nw + 2], args[nw + 3]

        idx = lax.broadcasted_iota(jnp.int32, (1, L), 1)
        row = idx // W
        col = lax.rem(idx, W)

        masks = {}

        def tap_mask(dy, dx):
            if (dy, dx) not in masks:
                conds = []
                if dy > 0:
                    conds.append(row < H - dy)
                elif dy < 0:
                    conds.append(row >= -dy)
                if dx > 0:
                    conds.append(col < W - dx)
                elif dx < 0:
                    conds.append(col >= -dx)
                m = conds[0]
                for c in conds[1:]:
                    m = m & c
                masks[(dy, dx)] = m
            return masks[(dy, dx)]

        def shift2d(x, dy, dx):
            if dy == 0 and dx == 0:
                return x
            s = (-(dy * W + dx)) % L
            return jnp.where(tap_mask(dy, dx), pltpu.roll(x, s, axis=1),
                             jnp.zeros((), bf16))

        def wb(name):
            g, r0, r1 = wslots[name]
            b0, b1 = bslots[name]
            return wrefs[g][r0:r1, :], ball_ref[b0:b1, :]

        def conv3x3(xb, name, dil, relu):
            w, b = wb(name)
            cin = xb.shape[0]
            r = 0
            for ky in (-1, 0, 1):
                for kx in (-1, 0, 1):
                    colbuf[r:r + cin, :] = shift2d(xb, ky * dil, kx * dil)
                    r += cin
            y = jnp.dot(w, colbuf[0:r, :],
                        preferred_element_type=jnp.float32) + b
            return jnp.maximum(y, 0.0) if relu else y

        xb = x_ref[0].astype(bf16)
        wcat, bcat = wb("wcat")
        t = jnp.dot(wcat, xb, preferred_element_type=jnp.float32) + bcat
        i = inter
        z0 = t[0:i].astype(bf16)
        z1 = t[i:2 * i].astype(bf16)
        z2 = t[2 * i:3 * i].astype(bf16)
        short = t[3 * i:]

        co = 2 * inter

        y0 = conv3x3(z0, "w01", 1, True).astype(bf16)
        ybuf[0:co, :] = conv3x3(y0, "w02", vision + 1, False).astype(bf16)
        y1 = conv3x3(z1, "w11", 1, True).astype(bf16)
        ybuf[co:2 * co, :] = conv3x3(y1, "w12", vision + 2, False).astype(bf16)
        y2 = conv3x3(z2, "w21", 1, True).astype(bf16)
        y2 = conv3x3(y2, "w22", 1, True).astype(bf16)
        ybuf[2 * co:3 * co, :] = conv3x3(y2, "w23", vision + 4, False).astype(bf16)

        wl, blin = wb("wl")
        out = jnp.dot(wl, ybuf[...], preferred_element_type=jnp.float32) + blin

        out = out * scale + short
        o_ref[0] = jnp.maximum(out, 0.0).astype(o_ref.dtype)

    return body


@functools.partial(jax.jit, static_argnames=("scale", "vision"))
def _rfb_forward(x_nchw, weights, *, scale=0.1, vision=1):
    N, C1, H, W = x_nchw.shape
    L = H * W
    bf16 = jnp.bfloat16

    wcat = weights["wcat"]
    c2 = weights["blin"].shape[0]
    inter = (wcat.shape[0] - c2) // 3

    names = ["wcat", "w01", "w02", "w11", "w12", "w21", "w22", "w23", "wl"]
    bias_of = {"wcat": "bcat", "w01": "b01", "w02": "b02", "w11": "b11",
               "w12": "b12", "w21": "b21", "w22": "b22", "w23": "b23",
               "wl": "blin"}

    bywidth = {}
    for n in names:
        bywidth.setdefault(weights[n].shape[1], []).append(n)
    wslots, wops = {}, []
    for g, (k, group) in enumerate(sorted(bywidth.items())):
        r = 0
        parts = []
        for n in group:
            m = weights[n].shape[0]
            wslots[n] = (g, r, r + m)
            parts.append(weights[n].astype(bf16))
            r += m
        wops.append(parts[0] if len(parts) == 1
                    else jnp.concatenate(parts, axis=0))

    bslots, bparts = {}, []
    r = 0
    for n in names:
        b = weights[bias_of[n]]
        bslots[n] = (r, r + b.shape[0])
        bparts.append(b)
        r += b.shape[0]
    ball = jnp.concatenate(bparts, axis=0)

    xf = x_nchw.reshape(N, C1, L)
    kernel_fn = _make_rfb_kernel(H, W, inter, scale, vision, wslots, bslots)

    rows = 9 * max(2 * inter, inter // 2 * 3, inter)
    rows = -(-rows // 16) * 16

    full = lambda a: pl.BlockSpec(a.shape, lambda n, _z=(0,) * a.ndim: _z)
    out = pl.pallas_call(
        kernel_fn,
        out_shape=jax.ShapeDtypeStruct((N, c2, L), x_nchw.dtype),
        grid=(N,),
        in_specs=[pl.BlockSpec((1, C1, L), lambda n: (n, 0, 0))]
                 + [full(w) for w in wops] + [full(ball)],
        out_specs=pl.BlockSpec((1, c2, L), lambda n: (n, 0, 0)),
        scratch_shapes=[pltpu.VMEM((rows, L), bf16),
                        pltpu.VMEM((3 * 2 * inter, L), bf16)],
        compiler_params=pltpu.CompilerParams(dimension_semantics=("parallel",)),
    )(xf, *wops, ball)
    return out.reshape(N, c2, H, W)


def kernel(x, wcat, bcat, b0_w0, b0_b0, b0_w1, b0_b1,
           b1_w0, b1_b0, b1_w1, b1_b1,
           b2_w0, b2_b0, b2_w1, b2_b1, b2_w2, b2_b2,
           wl0, wl1, wl2, blin):
    weights = {
        "wcat": wcat, "bcat": bcat,
        "w01": b0_w0, "b01": b0_b0, "w02": b0_w1, "b02": b0_b1,
        "w11": b1_w0, "b11": b1_b0, "w12": b1_w1, "b12": b1_b1,
        "w21": b2_w0, "b21": b2_b0, "w22": b2_w1, "b22": b2_b1,
        "w23": b2_w2, "b23": b2_b2,
        "wl": jnp.concatenate([wl0, wl1, wl2], axis=1), "blin": blin,
    }
    return _rfb_forward(x, weights, scale=0.1, vision=1)


# speedup vs baseline: 1.2357x; 1.0948x over previous
import functools

import jax
import jax.numpy as jnp
from jax import lax
from jax.experimental import pallas as pl
from jax.experimental.pallas import tpu as pltpu


def _make_rfb_kernel(H, W, inter, scale, vision, wslots, bslots):
    L = H * W
    bf16 = jnp.bfloat16

    def body(x_ref, *args):
        nw = len(set(g for g, _, _ in wslots.values()))
        wrefs = args[:nw]
        ball_ref = args[nw]
        o_ref, colbuf, ybuf = args[nw + 1], args[nw + 2], args[nw + 3]

        idx = lax.broadcasted_iota(jnp.int32, (1, L), 1)
        row = idx // W
        col = lax.rem(idx, W)

        masks = {}

        def tap_mask(dy, dx):
            if (dy, dx) not in masks:
                conds = []
                if dy > 0:
                    conds.append(row < H - dy)
                elif dy < 0:
                    conds.append(row >= -dy)
                if dx > 0:
                    conds.append(col < W - dx)
                elif dx < 0:
                    conds.append(col >= -dx)
                m = conds[0]
                for c in conds[1:]:
                    m = m & c
                masks[(dy, dx)] = m
            return masks[(dy, dx)]

        def shift2d(x, dy, dx):
            if dy == 0 and dx == 0:
                return x
            s = (-(dy * W + dx)) % L
            return jnp.where(tap_mask(dy, dx), pltpu.roll(x, s, axis=1),
                             jnp.zeros((), bf16))

        def wb(name):
            g, r0, r1 = wslots[name]
            b0, b1 = bslots[name]
            return wrefs[g][r0:r1, :], ball_ref[b0:b1, :]

        base = [0]

        def conv3x3(xb, name, dil, relu):
            w, b = wb(name)
            cin = xb.shape[0]
            r0 = r = base[0]
            for ky in (-1, 0, 1):
                for kx in (-1, 0, 1):
                    colbuf[r:r + cin, :] = shift2d(xb, ky * dil, kx * dil)
                    r += cin
            base[0] = r
            y = jnp.dot(w, colbuf[r0:r, :],
                        preferred_element_type=jnp.float32) + b
            return jnp.maximum(y, 0.0) if relu else y

        xb = x_ref[0].astype(bf16)
        wcat, bcat = wb("wcat")
        t = jnp.dot(wcat, xb, preferred_element_type=jnp.float32) + bcat
        i = inter
        z0 = t[0:i].astype(bf16)
        z1 = t[i:2 * i].astype(bf16)
        z2 = t[2 * i:3 * i].astype(bf16)
        short = t[3 * i:]

        co = 2 * inter

        y0 = conv3x3(z0, "w01", 1, True).astype(bf16)
        y1 = conv3x3(z1, "w11", 1, True).astype(bf16)
        y2 = conv3x3(z2, "w21", 1, True).astype(bf16)
        y2 = conv3x3(y2, "w22", 1, True).astype(bf16)
        ybuf[0:co, :] = conv3x3(y0, "w02", vision + 1, False).astype(bf16)
        ybuf[co:2 * co, :] = conv3x3(y1, "w12", vision + 2, False).astype(bf16)
        ybuf[2 * co:3 * co, :] = conv3x3(y2, "w23", vision + 4, False).astype(bf16)

        wl, blin = wb("wl")
        out = jnp.dot(wl, ybuf[...], preferred_element_type=jnp.float32) + blin

        out = out * scale + short
        o_ref[0] = jnp.maximum(out, 0.0).astype(o_ref.dtype)

    return body


@functools.partial(jax.jit, static_argnames=("scale", "vision"))
def _rfb_forward(x_nchw, weights, *, scale=0.1, vision=1):
    N, C1, H, W = x_nchw.shape
    L = H * W
    bf16 = jnp.bfloat16

    wcat = weights["wcat"]
    c2 = weights["blin"].shape[0]
    inter = (wcat.shape[0] - c2) // 3

    names = ["wcat", "w01", "w02", "w11", "w12", "w21", "w22", "w23", "wl"]
    bias_of = {"wcat": "bcat", "w01": "b01", "w02": "b02", "w11": "b11",
               "w12": "b12", "w21": "b21", "w22": "b22", "w23": "b23",
               "wl": "blin"}

    bywidth = {}
    for n in names:
        bywidth.setdefault(weights[n].shape[1], []).append(n)
    wslots, wops = {}, []
    for g, (k, group) in enumerate(sorted(bywidth.items())):
        r = 0
        parts = []
        for n in group:
            m = weights[n].shape[0]
            wslots[n] = (g, r, r + m)
            parts.append(weights[n].astype(bf16))
            r += m
        wops.append(parts[0] if len(parts) == 1
                    else jnp.concatenate(parts, axis=0))

    bslots, bparts = {}, []
    r = 0
    for n in names:
        b = weights[bias_of[n]]
        bslots[n] = (r, r + b.shape[0])
        bparts.append(b)
        r += b.shape[0]
    ball = jnp.concatenate(bparts, axis=0)

    xf = x_nchw.reshape(N, C1, L)
    kernel_fn = _make_rfb_kernel(H, W, inter, scale, vision, wslots, bslots)

    rows = 9 * (inter * 3 + inter // 2 * 3 + 2 * inter * 4)
    rows = -(-rows // 16) * 16

    full = lambda a: pl.BlockSpec(a.shape, lambda n, _z=(0,) * a.ndim: _z)
    out = pl.pallas_call(
        kernel_fn,
        out_shape=jax.ShapeDtypeStruct((N, c2, L), x_nchw.dtype),
        grid=(N,),
        in_specs=[pl.BlockSpec((1, C1, L), lambda n: (n, 0, 0))]
                 + [full(w) for w in wops] + [full(ball)],
        out_specs=pl.BlockSpec((1, c2, L), lambda n: (n, 0, 0)),
        scratch_shapes=[pltpu.VMEM((rows, L), bf16),
                        pltpu.VMEM((3 * 2 * inter, L), bf16)],
        compiler_params=pltpu.CompilerParams(dimension_semantics=("parallel",)),
    )(xf, *wops, ball)
    return out.reshape(N, c2, H, W)


def kernel(x, wcat, bcat, b0_w0, b0_b0, b0_w1, b0_b1,
           b1_w0, b1_b0, b1_w1, b1_b1,
           b2_w0, b2_b0, b2_w1, b2_b1, b2_w2, b2_b2,
           wl0, wl1, wl2, blin):
    weights = {
        "wcat": wcat, "bcat": bcat,
        "w01": b0_w0, "b01": b0_b0, "w02": b0_w1, "b02": b0_b1,
        "w11": b1_w0, "b11": b1_b0, "w12": b1_w1, "b12": b1_b1,
        "w21": b2_w0, "b21": b2_b0, "w22": b2_w1, "b22": b2_b1,
        "w23": b2_w2, "b23": b2_b2,
        "wl": jnp.concatenate([wl0, wl1, wl2], axis=1), "blin": blin,
    }
    return _rfb_forward(x, weights, scale=0.1, vision=1)


# speedup vs baseline: 1.2511x; 1.0124x over previous
import functools

import jax
import jax.numpy as jnp
from jax import lax
from jax.experimental import pallas as pl
from jax.experimental.pallas import tpu as pltpu


def _make_rfb_kernel(H, W, inter, scale, vision, wslots, bslots, nw):
    L = H * W
    bf16 = jnp.bfloat16

    def body(x_ref, *args):
        wrefs = args[:nw]
        ball_ref = args[nw]
        o_ref, colbuf, ybuf = args[nw + 1], args[nw + 2], args[nw + 3]

        idx = lax.broadcasted_iota(jnp.int32, (1, L), 1)
        row = idx // W
        col = lax.rem(idx, W)

        masks = {}

        def tap_mask(dy, dx):
            if (dy, dx) not in masks:
                conds = []
                if dy > 0:
                    conds.append(row < H - dy)
                elif dy < 0:
                    conds.append(row >= -dy)
                if dx > 0:
                    conds.append(col < W - dx)
                elif dx < 0:
                    conds.append(col >= -dx)
                m = conds[0]
                for c in conds[1:]:
                    m = m & c
                masks[(dy, dx)] = m
            return masks[(dy, dx)]

        def shift2d(x, dy, dx):
            if dy == 0 and dx == 0:
                return x
            s = (-(dy * W + dx)) % L
            return jnp.where(tap_mask(dy, dx), pltpu.roll(x, s, axis=1),
                             jnp.zeros((), bf16))

        def wb(name):
            g, r0, r1 = wslots[name]
            b0, b1 = bslots[name]
            return wrefs[g][r0:r1, :], ball_ref[b0:b1, :]

        base = [0]

        def conv3x3(xb, name, dil, relu):
            w, b = wb(name)
            cin = xb.shape[0]
            r0 = r = base[0]
            for ky in (-1, 0, 1):
                for kx in (-1, 0, 1):
                    colbuf[r:r + cin, :] = shift2d(xb, ky * dil, kx * dil)
                    r += cin
            base[0] = r
            y = jnp.dot(w, colbuf[r0:r, :],
                        preferred_element_type=jnp.float32) + b
            return jnp.maximum(y, 0.0) if relu else y

        def one_image(img):
            xb = x_ref[img].astype(bf16)
            wcat, bcat = wb("wcat")
            i = inter
            t = jnp.dot(wcat[0:3 * i], xb,
                        preferred_element_type=jnp.float32) + bcat[0:3 * i]
            z0 = t[0:i].astype(bf16)
            z1 = t[i:2 * i].astype(bf16)
            z2 = t[2 * i:3 * i].astype(bf16)
            short = jnp.dot(wcat[3 * i:], xb,
                            preferred_element_type=jnp.float32) + bcat[3 * i:]

            co = 2 * inter

            y0 = conv3x3(z0, "w01", 1, True).astype(bf16)
            y1 = conv3x3(z1, "w11", 1, True).astype(bf16)
            y2 = conv3x3(z2, "w21", 1, True).astype(bf16)
            y2 = conv3x3(y2, "w22", 1, True).astype(bf16)
            yb = ybuf.at[img]
            yb[0:co, :] = conv3x3(y0, "w02", vision + 1, False).astype(bf16)
            yb[co:2 * co, :] = conv3x3(y1, "w12", vision + 2,
                                       False).astype(bf16)
            yb[2 * co:3 * co, :] = conv3x3(y2, "w23", vision + 4,
                                           False).astype(bf16)

            wl, blin = wb("wl")
            out = jnp.dot(wl, yb[...],
                          preferred_element_type=jnp.float32) + blin

            out = out * scale + short
            o_ref[img] = jnp.maximum(out, 0.0).astype(o_ref.dtype)

        for img in range(x_ref.shape[0]):
            one_image(img)

    return body


@functools.partial(jax.jit, static_argnames=("scale", "vision"))
def _rfb_forward(x_nchw, weights, *, scale=0.1, vision=1):
    N, C1, H, W = x_nchw.shape
    L = H * W
    bf16 = jnp.bfloat16

    wcat = weights["wcat"]
    c2 = weights["blin"].shape[0]
    inter = (wcat.shape[0] - c2) // 3

    names = ["wcat", "w01", "w02", "w11", "w12", "w21", "w22", "w23", "wl"]
    bias_of = {"wcat": "bcat", "w01": "b01", "w02": "b02", "w11": "b11",
               "w12": "b12", "w21": "b21", "w22": "b22", "w23": "b23",
               "wl": "blin"}

    bywidth = {}
    for n in names:
        bywidth.setdefault(weights[n].shape[1], []).append(n)
    wslots, wops = {}, []
    for g, (k, group) in enumerate(sorted(bywidth.items())):
        r = 0
        parts = []
        for n in group:
            m = weights[n].shape[0]
            wslots[n] = (g, r, r + m)
            parts.append(weights[n].astype(bf16))
            r += m
        wops.append(parts[0] if len(parts) == 1
                    else jnp.concatenate(parts, axis=0))

    bslots, bparts = {}, []
    r = 0
    for n in names:
        b = weights[bias_of[n]]
        bslots[n] = (r, r + b.shape[0])
        bparts.append(b)
        r += b.shape[0]
    ball = jnp.concatenate(bparts, axis=0)

    xf = x_nchw.reshape(N, C1, L)
    kernel_fn = _make_rfb_kernel(H, W, inter, scale, vision, wslots, bslots,
                                 len(wops))

    B = 2
    rows = B * 9 * (inter * 3 + inter // 2 * 3 + 2 * inter * 4)
    rows = -(-rows // 16) * 16

    full = lambda a: pl.BlockSpec(a.shape, lambda n, _z=(0,) * a.ndim: _z)
    out = pl.pallas_call(
        kernel_fn,
        out_shape=jax.ShapeDtypeStruct((N, c2, L), x_nchw.dtype),
        grid=(N // B,),
        in_specs=[pl.BlockSpec((B, C1, L), lambda n: (n, 0, 0))]
                 + [full(w) for w in wops] + [full(ball)],
        out_specs=pl.BlockSpec((B, c2, L), lambda n: (n, 0, 0)),
        scratch_shapes=[pltpu.VMEM((rows, L), bf16),
                        pltpu.VMEM((B, 3 * 2 * inter, L), bf16)],
        compiler_params=pltpu.CompilerParams(dimension_semantics=("parallel",)),
    )(xf, *wops, ball)
    return out.reshape(N, c2, H, W)


def kernel(x, wcat, bcat, b0_w0, b0_b0, b0_w1, b0_b1,
           b1_w0, b1_b0, b1_w1, b1_b1,
           b2_w0, b2_b0, b2_w1, b2_b1, b2_w2, b2_b2,
           wl0, wl1, wl2, blin):
    weights = {
        "wcat": wcat, "bcat": bcat,
        "w01": b0_w0, "b01": b0_b0, "w02": b0_w1, "b02": b0_b1,
        "w11": b1_w0, "b11": b1_b0, "w12": b1_w1, "b12": b1_b1,
        "w21": b2_w0, "b21": b2_b0, "w22": b2_w1, "b22": b2_b1,
        "w23": b2_w2, "b23": b2_b2,
        "wl": jnp.concatenate([wl0, wl1, wl2], axis=1), "blin": blin,
    }
    return _rfb_forward(x, weights, scale=0.1, vision=1)
